# Initial kernel scaffold; baseline (speedup 1.0000x reference)
import jax
import jax.numpy as jnp
from jax.experimental import pallas as pl


def kernel(sentences, words_per_sentence, table):
    raise NotImplementedError("write your pallas kernel here")


# speedup vs baseline: 2.4840x; 2.4840x over previous
import functools

import jax
import jax.numpy as jnp
from jax import lax
from jax.experimental import pallas as pl
from jax.experimental.pallas import tpu as pltpu
from jax.experimental.pallas import tpu_sc as plsc

VOCAB_SIZE = 1000000
EMB = 64
BATCH = 16384
PAD = 50

NUM_CORES = 2
NUM_SUBCORES = 16
LANES = 16
NW = NUM_CORES * NUM_SUBCORES

SENT_PER_W = BATCH // NW
CH = 16
NCHUNK = SENT_PER_W // CH
IDX_COLS = 100
ROWS_PER_CHUNK = CH * PAD
GATHERS = ROWS_PER_CHUNK // IDX_COLS
IDX_ROWS_PER_W = SENT_PER_W * PAD // IDX_COLS

_mesh = plsc.VectorSubcoreMesh(core_axis_name="c", subcore_axis_name="s")


@functools.partial(
    pl.kernel,
    out_type=jax.ShapeDtypeStruct((BATCH, EMB), jnp.float32),
    mesh=_mesh,
    compiler_params=pltpu.CompilerParams(use_tc_tiling_on_sc=False),
    scratch_types=[
        pltpu.VMEM((GATHERS, IDX_COLS), jnp.int32),
        pltpu.VMEM((ROWS_PER_CHUNK, EMB), jnp.float32),
        pltpu.VMEM((CH, EMB), jnp.float32),
        pltpu.SemaphoreType.DMA,
    ],
)
def _bag(sent_hbm, table_hbm, out_hbm, idx_v, rows_v, out_v, sem):
    wid = lax.axis_index("s") * NUM_CORES + lax.axis_index("c")

    def chunk_body(c, carry):
        irow0 = wid * IDX_ROWS_PER_W + c * GATHERS
        pltpu.sync_copy(sent_hbm.at[pl.ds(irow0, GATHERS)], idx_v)
        copies = []
        for m in range(GATHERS):
            copies.append(pltpu.async_copy(
                table_hbm.at[idx_v.at[m]],
                rows_v.at[pl.ds(m * IDX_COLS, IDX_COLS)],
                sem))
        for cp in copies:
            cp.wait()

        def sent_body(s, carry2):
            base = s * PAD
            acc = [rows_v[base, pl.ds(q * LANES, LANES)]
                   for q in range(EMB // LANES)]
            for j in range(1, PAD):
                for q in range(EMB // LANES):
                    acc[q] = acc[q] + rows_v[base + j, pl.ds(q * LANES, LANES)]
            for q in range(EMB // LANES):
                out_v[s, pl.ds(q * LANES, LANES)] = acc[q]
            return carry2

        lax.fori_loop(0, CH, sent_body, 0)
        orow0 = wid * SENT_PER_W + c * CH
        pltpu.sync_copy(out_v, out_hbm.at[pl.ds(orow0, CH)])
        return carry

    lax.fori_loop(0, NCHUNK, chunk_body, 0)


def kernel(sentences, words_per_sentence, table):
    del words_per_sentence
    sent_rows = sentences.reshape(BATCH * PAD // IDX_COLS, IDX_COLS)
    return _bag(sent_rows, table)


# speedup vs baseline: 2.7435x; 1.1045x over previous
import functools

import jax
import jax.numpy as jnp
from jax import lax
from jax.experimental import pallas as pl
from jax.experimental.pallas import tpu as pltpu
from jax.experimental.pallas import tpu_sc as plsc

VOCAB_SIZE = 1000000
EMB = 64
BATCH = 16384
PAD = 50

NUM_CORES = 2
NUM_SUBCORES = 16
LANES = 16
NW = NUM_CORES * NUM_SUBCORES

SENT_PER_W = BATCH // NW
CH = 16
NCHUNK = SENT_PER_W // CH
IDX_COLS = 100
ROWS_PER_CHUNK = CH * PAD
GATHERS = ROWS_PER_CHUNK // IDX_COLS
IDX_ROWS_PER_W = SENT_PER_W * PAD // IDX_COLS

_mesh = plsc.VectorSubcoreMesh(core_axis_name="c", subcore_axis_name="s")


@functools.partial(
    pl.kernel,
    out_type=jax.ShapeDtypeStruct((BATCH, EMB), jnp.float32),
    mesh=_mesh,
    compiler_params=pltpu.CompilerParams(use_tc_tiling_on_sc=False),
    scratch_types=[
        pltpu.VMEM((2, GATHERS, IDX_COLS), jnp.int32),
        pltpu.VMEM((2, ROWS_PER_CHUNK, EMB), jnp.float32),
        pltpu.VMEM((CH, EMB), jnp.float32),
        pltpu.SemaphoreType.DMA((2,)),
    ],
)
def _bag(sent_hbm, table_hbm, out_hbm, idx_v, rows_v, out_v, sem):
    wid = lax.axis_index("s") * NUM_CORES + lax.axis_index("c")

    def fire_chunk(c, buf):
        irow0 = wid * IDX_ROWS_PER_W + c * GATHERS
        pltpu.sync_copy(sent_hbm.at[pl.ds(irow0, GATHERS)], idx_v.at[buf])
        for m in range(GATHERS):
            pltpu.async_copy(
                table_hbm.at[idx_v.at[buf, m]],
                rows_v.at[buf].at[pl.ds(m * IDX_COLS, IDX_COLS)],
                sem.at[buf])

    fire_chunk(0, 0)

    def chunk_body(c, carry):
        cur = lax.rem(c, 2)
        nxt = 1 - cur

        @pl.when(c + 1 < NCHUNK)
        def _():
            fire_chunk(c + 1, nxt)

        pltpu.make_async_copy(
            table_hbm.at[pl.ds(0, ROWS_PER_CHUNK)],
            rows_v.at[cur],
            sem.at[cur]).wait()

        def sent_body(s, carry2):
            base = s * PAD
            acc = [rows_v[cur, base, pl.ds(q * LANES, LANES)]
                   for q in range(EMB // LANES)]
            for j in range(1, PAD):
                for q in range(EMB // LANES):
                    acc[q] = acc[q] + rows_v[cur, base + j,
                                             pl.ds(q * LANES, LANES)]
            for q in range(EMB // LANES):
                out_v[s, pl.ds(q * LANES, LANES)] = acc[q]
            return carry2

        lax.fori_loop(0, CH, sent_body, 0)
        orow0 = wid * SENT_PER_W + c * CH
        pltpu.sync_copy(out_v, out_hbm.at[pl.ds(orow0, CH)])
        return carry

    lax.fori_loop(0, NCHUNK, chunk_body, 0)


def kernel(sentences, words_per_sentence, table):
    del words_per_sentence
    sent_rows = sentences.reshape(BATCH * PAD // IDX_COLS, IDX_COLS)
    return _bag(sent_rows, table)
